# baseline (speedup 1.0000x reference)
import functools

import jax
import jax.numpy as jnp
from jax import lax
from jax.experimental import pallas as pl
from jax.experimental.pallas import tpu as pltpu
from jax.experimental.pallas import tpu_sc as plsc

IN_SIZE = 2048
OUT_SIZE = 2048
BATCH = 8192
NNZ = IN_SIZE * OUT_SIZE // 2

NS = 16
HALF_COLS = OUT_SIZE // 2
CHUNK = IN_SIZE * HALF_COLS // 2
DUMP = 2048
PER_TILE = NNZ // NS
WSZ = 4096
NWIN = PER_TILE // WSZ
JROWS = WSZ // 128
SLICE = CHUNK // NS


CW = WSZ + 256
MAXROWS = CW // 128


def _sc_scatter_body(col_base, flat_ref, val_ref, out_ref,
                     spmem, flatw, valw, cloc, cval, locb, drainb,
                     sem, in_sem):
    c = lax.axis_index("c")
    s = lax.axis_index("s")
    z16 = jnp.zeros((16,), jnp.float32)
    ii16 = lax.iota(jnp.int32, 16)
    tile_base = s * PER_TILE
    sel_mask = (1 << 21) | (1 << 10)
    want = c * (1 << 21) + (col_base // HALF_COLS) * (1 << 10)
    dump16 = CHUNK + s * 128 + ii16 * 8

    def prefetch(w):
        start = tile_base + w * WSZ
        pltpu.async_copy(flat_ref.at[pl.ds(start, WSZ)], flatw.at[w % 2],
                         in_sem)
        pltpu.async_copy(val_ref.at[pl.ds(start, WSZ)], valw.at[w % 2],
                         in_sem)

    def drain_rows(n):
        def dr(i, _):
            pltpu.make_async_copy(val_ref.at[pl.ds(0, 128)],
                                  drainb, sem).wait()
            return 0
        lax.fori_loop(0, n, dr, 0)

    def fire_row(f):
        for k in range(8):
            locb[f, pl.ds(k * 16, 16)] = cloc[pl.ds(f * 128 + k * 16, 16)]
        pltpu.async_copy(cval.at[pl.ds(f * 128, 128)],
                         spmem.at[locb.at[f]], sem, add=True)

    def zfill(i, _):
        valw[0, pl.ds(i * 16, 16)] = z16
        return 0
    lax.fori_loop(0, WSZ // 16, zfill, 0)
    for t in range(SLICE // WSZ):
        pltpu.sync_copy(valw.at[0],
                        spmem.at[pl.ds(s * SLICE + t * WSZ, WSZ)])
    plsc.subcore_barrier()

    prefetch(0)

    def window(w, carry):
        tail, r1 = carry
        s2 = w % 2

        @pl.when(w + 1 < NWIN)
        def _():
            prefetch(w + 1)

        start = tile_base + w * WSZ
        pltpu.make_async_copy(flat_ref.at[pl.ds(start, WSZ)],
                              flatw.at[s2], in_sem).wait()
        pltpu.make_async_copy(val_ref.at[pl.ds(start, WSZ)],
                              valw.at[s2], in_sem).wait()

        drain_rows(r1)

        @pl.when(tail > 0)
        def _():
            pbase = (r1 * 128).astype(jnp.int32)
            for k in range(8):
                cloc[pl.ds(k * 16, 16)] = cloc[pl.ds(pbase + k * 16, 16)]
                cval[pl.ds(k * 16, 16)] = cval[pl.ds(pbase + k * 16, 16)]

        def group(j, cnt):
            for k in range(8):
                e = j * 128 + k * 16
                flat = flatw[s2, pl.ds(e, 16)]
                v = valw[s2, pl.ds(e, 16)]
                mine = (flat & sel_mask) == want
                loc = ((flat >> 1) & 0x000FFC00) | (flat & 1023)
                plsc.store_compressed(cloc.at[pl.ds(cnt, 16)], loc,
                                      mask=mine)
                plsc.store_compressed(cval.at[pl.ds(cnt, 16)], v,
                                      mask=mine)
                npop = plsc.all_reduce_population_count(mine)
                cnt = cnt + npop[0]
            return cnt
        total = lax.fori_loop(0, JROWS, group, tail)

        rows = total >> 7

        def fire(f, _):
            fire_row(f)
            return 0
        lax.fori_loop(0, rows, fire, 0)
        return (total & 127, rows)
    tail, r1 = lax.fori_loop(0, NWIN, window,
                             (jnp.int32(0), jnp.int32(0)))

    base = r1 * 128 + tail

    @pl.when(tail > 0)
    def _():
        for k in range(8):
            cloc[pl.ds(base + k * 16, 16)] = dump16
            cval[pl.ds(base + k * 16, 16)] = z16
        fire_row(r1)
    last = jnp.where(tail > 0, 1, 0).astype(jnp.int32)
    drain_rows(r1 + last)
    plsc.subcore_barrier()

    pltpu.sync_copy(spmem.at[pl.ds(s * SLICE, SLICE)],
                    out_ref.at[pl.ds(c * CHUNK + s * SLICE, SLICE)])
    plsc.subcore_barrier()


@functools.partial(jax.jit, static_argnums=0)
def _sc_scatter(col_base, flat, val):
    mesh = plsc.VectorSubcoreMesh(core_axis_name="c", subcore_axis_name="s")
    return pl.kernel(
        functools.partial(_sc_scatter_body, col_base),
        out_type=jax.ShapeDtypeStruct((IN_SIZE * HALF_COLS,), jnp.float32),
        mesh=mesh,
        scratch_types=[
            pltpu.VMEM_SHARED((CHUNK + DUMP,), jnp.float32),
            pltpu.VMEM((2, WSZ), jnp.int32),
            pltpu.VMEM((2, WSZ), jnp.float32),
            pltpu.VMEM((CW,), jnp.int32),
            pltpu.VMEM((CW,), jnp.float32),
            pltpu.VMEM((MAXROWS, 128), jnp.int32),
            pltpu.VMEM((128,), jnp.float32),
            pltpu.SemaphoreType.DMA,
            pltpu.SemaphoreType.DMA,
        ],
        compiler_params=pltpu.CompilerParams(needs_layout_passes=False),
        cost_estimate=pl.CostEstimate(flops=0, transcendentals=0,
                                      bytes_accessed=400_000_000),
    )(flat, val)


BM = 512


def _mm_body_first(x_ref, w_ref, b_ref, o_ref):
    acc = jax.lax.dot_general(x_ref[...], w_ref[...],
                              (((1,), (0,)), ((), ())),
                              preferred_element_type=jnp.float32)
    o_ref[...] = jnp.maximum(acc + b_ref[...], 0.0)


def _mm_body_second(x_ref, w_ref, b_ref, z_ref, o_ref):
    del z_ref
    acc = jax.lax.dot_general(x_ref[...], w_ref[...],
                              (((1,), (0,)), ((), ())),
                              preferred_element_type=jnp.float32)
    o_ref[...] = jnp.maximum(acc + b_ref[...], 0.0)


@jax.jit
def _matmul_half_first(x_bf, w_bf, bias_half):
    return pl.pallas_call(
        _mm_body_first,
        grid=(BATCH // BM,),
        in_specs=[
            pl.BlockSpec((BM, IN_SIZE), lambda i: (i, 0)),
            pl.BlockSpec((IN_SIZE, HALF_COLS), lambda i: (0, 0)),
            pl.BlockSpec((1, HALF_COLS), lambda i: (0, 0)),
        ],
        out_specs=pl.BlockSpec((BM, HALF_COLS), lambda i: (i, 0)),
        out_shape=jax.ShapeDtypeStruct((BATCH, OUT_SIZE), jnp.float32),
    )(x_bf, w_bf, bias_half)


@jax.jit
def _matmul_half_second(x_bf, w_bf, bias_half, z_prev):
    return pl.pallas_call(
        _mm_body_second,
        grid=(BATCH // BM,),
        in_specs=[
            pl.BlockSpec((BM, IN_SIZE), lambda i: (i, 0)),
            pl.BlockSpec((IN_SIZE, HALF_COLS), lambda i: (0, 0)),
            pl.BlockSpec((1, HALF_COLS), lambda i: (0, 0)),
            pl.BlockSpec(memory_space=pl.ANY),
        ],
        out_specs=pl.BlockSpec((BM, HALF_COLS), lambda i: (i, 1)),
        out_shape=jax.ShapeDtypeStruct((BATCH, OUT_SIZE), jnp.float32),
        input_output_aliases={3: 0},
    )(x_bf, w_bf, bias_half, z_prev)


def kernel(x, idx, val, bias):
    idx32 = idx.astype(jnp.int32)
    flat = idx32[:, 0] * OUT_SIZE + idx32[:, 1]
    x_bf = x.astype(jnp.bfloat16)
    bias2 = bias.reshape(2, HALF_COLS)

    wa = _sc_scatter(0, flat, val)
    wb = _sc_scatter(HALF_COLS, flat, val)
    wa_bf = wa.reshape(IN_SIZE, HALF_COLS).astype(jnp.bfloat16)
    wb_bf = wb.reshape(IN_SIZE, HALF_COLS).astype(jnp.bfloat16)

    z = _matmul_half_first(x_bf, wa_bf, bias2[0].reshape(1, HALF_COLS))
    z = _matmul_half_second(x_bf, wb_bf, bias2[1].reshape(1, HALF_COLS), z)
    return z


# speedup vs baseline: 1.0793x; 1.0793x over previous
import functools

import jax
import jax.numpy as jnp
from jax import lax
from jax.experimental import pallas as pl
from jax.experimental.pallas import tpu as pltpu
from jax.experimental.pallas import tpu_sc as plsc

IN_SIZE = 2048
OUT_SIZE = 2048
BATCH = 8192
NNZ = IN_SIZE * OUT_SIZE // 2

NS = 16
HALF_COLS = OUT_SIZE // 2
CHUNK = IN_SIZE * HALF_COLS // 2
DUMP = 2048
PER_TILE = NNZ // NS
WSZ = 4096
NWIN = PER_TILE // WSZ
JROWS = WSZ // 128
SLICE = CHUNK // NS


def _sc_scatter_body(col_base, flat_ref, val_ref, out_ref,
                     spmem, flatw, valw, locb, sem, in_sem):
    c = lax.axis_index("c")
    s = lax.axis_index("s")
    z16 = jnp.zeros((16,), jnp.float32)
    tile_base = s * PER_TILE
    row_base = c * (IN_SIZE // 2)

    def prefetch(w):
        start = tile_base + w * WSZ
        pltpu.async_copy(flat_ref.at[pl.ds(start, WSZ)], flatw.at[w % 2],
                         in_sem)
        pltpu.async_copy(val_ref.at[pl.ds(start, WSZ)], valw.at[w % 3],
                         in_sem)

    def drain_scatters(slot3):
        pltpu.make_async_copy(val_ref.at[pl.ds(0, WSZ)],
                              valw.at[slot3], sem).wait()

    def zfill(i, _):
        valw[0, pl.ds(i * 16, 16)] = z16
        return 0
    lax.fori_loop(0, WSZ // 16, zfill, 0)
    for t in range(SLICE // WSZ):
        pltpu.sync_copy(valw.at[0],
                        spmem.at[pl.ds(s * SLICE + t * WSZ, WSZ)])
    plsc.subcore_barrier()

    prefetch(0)

    def window(w, _):
        s2 = w % 2
        s3 = w % 3

        @pl.when(w >= 2)
        def _():
            drain_scatters((w + 1) % 3)

        @pl.when(w + 1 < NWIN)
        def _():
            prefetch(w + 1)

        start = tile_base + w * WSZ
        pltpu.make_async_copy(flat_ref.at[pl.ds(start, WSZ)],
                              flatw.at[s2], in_sem).wait()
        pltpu.make_async_copy(val_ref.at[pl.ds(start, WSZ)],
                              valw.at[s3], in_sem).wait()

        def group(j, _):
            for k in range(8):
                e = j * 128 + k * 16
                flat = flatw[s2, pl.ds(e, 16)]
                rl = (flat >> 11) - row_base
                cl = (flat & (OUT_SIZE - 1)) - col_base
                mine = (rl.astype(jnp.uint32) < HALF_COLS) & (
                    cl.astype(jnp.uint32) < HALF_COLS)
                loc = jnp.where(mine, rl * HALF_COLS + cl,
                                CHUNK + (flat & (DUMP - 1)))
                locb[s3, j, pl.ds(k * 16, 16)] = loc
            return 0
        lax.fori_loop(0, JROWS, group, 0)

        def fire(j, _):
            pltpu.async_copy(valw.at[s3, pl.ds(j * 128, 128)],
                             spmem.at[locb.at[s3, j]], sem, add=True)
            return 0
        lax.fori_loop(0, JROWS, fire, 0)
        return 0
    lax.fori_loop(0, NWIN, window, 0)

    for t in range(2):
        drain_scatters(t)
    plsc.subcore_barrier()

    pltpu.sync_copy(spmem.at[pl.ds(s * SLICE, SLICE)],
                    out_ref.at[pl.ds(c * CHUNK + s * SLICE, SLICE)])
    plsc.subcore_barrier()


@functools.partial(jax.jit, static_argnums=0)
def _sc_scatter(col_base, flat, val):
    mesh = plsc.VectorSubcoreMesh(core_axis_name="c", subcore_axis_name="s")
    return pl.kernel(
        functools.partial(_sc_scatter_body, col_base),
        out_type=jax.ShapeDtypeStruct((IN_SIZE * HALF_COLS,), jnp.float32),
        mesh=mesh,
        scratch_types=[
            pltpu.VMEM_SHARED((CHUNK + DUMP,), jnp.float32),
            pltpu.VMEM((2, WSZ), jnp.int32),
            pltpu.VMEM((3, WSZ), jnp.float32),
            pltpu.VMEM((3, JROWS, 128), jnp.int32),
            pltpu.SemaphoreType.DMA,
            pltpu.SemaphoreType.DMA,
        ],
        compiler_params=pltpu.CompilerParams(needs_layout_passes=False),
        cost_estimate=pl.CostEstimate(flops=0, transcendentals=0,
                                      bytes_accessed=400_000_000),
    )(flat, val)


BM = 512


def _mm_body_first(x_ref, w_ref, b_ref, o_ref):
    acc = jax.lax.dot_general(x_ref[...], w_ref[...],
                              (((1,), (0,)), ((), ())),
                              preferred_element_type=jnp.float32)
    o_ref[...] = jnp.maximum(acc + b_ref[...], 0.0)


def _mm_body_second(x_ref, w_ref, b_ref, z_ref, o_ref):
    del z_ref
    acc = jax.lax.dot_general(x_ref[...], w_ref[...],
                              (((1,), (0,)), ((), ())),
                              preferred_element_type=jnp.float32)
    o_ref[...] = jnp.maximum(acc + b_ref[...], 0.0)


@jax.jit
def _matmul_half_first(x_bf, w_bf, bias_half):
    return pl.pallas_call(
        _mm_body_first,
        grid=(BATCH // BM,),
        in_specs=[
            pl.BlockSpec((BM, IN_SIZE), lambda i: (i, 0)),
            pl.BlockSpec((IN_SIZE, HALF_COLS), lambda i: (0, 0)),
            pl.BlockSpec((1, HALF_COLS), lambda i: (0, 0)),
        ],
        out_specs=pl.BlockSpec((BM, HALF_COLS), lambda i: (i, 0)),
        out_shape=jax.ShapeDtypeStruct((BATCH, OUT_SIZE), jnp.float32),
    )(x_bf, w_bf, bias_half)


@jax.jit
def _matmul_half_second(x_bf, w_bf, bias_half, z_prev):
    return pl.pallas_call(
        _mm_body_second,
        grid=(BATCH // BM,),
        in_specs=[
            pl.BlockSpec((BM, IN_SIZE), lambda i: (i, 0)),
            pl.BlockSpec((IN_SIZE, HALF_COLS), lambda i: (0, 0)),
            pl.BlockSpec((1, HALF_COLS), lambda i: (0, 0)),
            pl.BlockSpec(memory_space=pl.ANY),
        ],
        out_specs=pl.BlockSpec((BM, HALF_COLS), lambda i: (i, 1)),
        out_shape=jax.ShapeDtypeStruct((BATCH, OUT_SIZE), jnp.float32),
        input_output_aliases={3: 0},
    )(x_bf, w_bf, bias_half, z_prev)


def kernel(x, idx, val, bias):
    idx32 = idx.astype(jnp.int32)
    flat = idx32[:, 0] * OUT_SIZE + idx32[:, 1]
    x_bf = x.astype(jnp.bfloat16)
    bias2 = bias.reshape(2, HALF_COLS)

    wa = _sc_scatter(0, flat, val)
    wb = _sc_scatter(HALF_COLS, flat, val)
    wa_bf = wa.reshape(IN_SIZE, HALF_COLS).astype(jnp.bfloat16)
    wb_bf = wb.reshape(IN_SIZE, HALF_COLS).astype(jnp.bfloat16)

    z = _matmul_half_first(x_bf, wa_bf, bias2[0].reshape(1, HALF_COLS))
    z = _matmul_half_second(x_bf, wb_bf, bias2[1].reshape(1, HALF_COLS), z)
    return z


# speedup vs baseline: 1.0798x; 1.0005x over previous
import functools

import jax
import jax.numpy as jnp
from jax import lax
from jax.experimental import pallas as pl
from jax.experimental.pallas import tpu as pltpu
from jax.experimental.pallas import tpu_sc as plsc

IN_SIZE = 2048
OUT_SIZE = 2048
BATCH = 8192
NNZ = IN_SIZE * OUT_SIZE // 2

NS = 16
HALF_COLS = OUT_SIZE // 2
CHUNK = IN_SIZE * HALF_COLS // 2
DUMP = 32768
PER_TILE = NNZ // NS
WSZ = 4096
NWIN = PER_TILE // WSZ
JROWS = WSZ // 128
SLICE = CHUNK // NS


def _sc_scatter_body(col_base, flat_ref, val_ref, out_ref,
                     spmem, flatw, valw, locb, sem, in_sem):
    c = lax.axis_index("c")
    s = lax.axis_index("s")
    z16 = jnp.zeros((16,), jnp.float32)
    tile_base = s * PER_TILE
    row_base = c * (IN_SIZE // 2)

    def prefetch(w):
        start = tile_base + w * WSZ
        pltpu.async_copy(flat_ref.at[pl.ds(start, WSZ)], flatw.at[w % 2],
                         in_sem)
        pltpu.async_copy(val_ref.at[pl.ds(start, WSZ)], valw.at[w % 3],
                         in_sem)

    def drain_scatters(slot3):
        pltpu.make_async_copy(val_ref.at[pl.ds(0, WSZ)],
                              valw.at[slot3], sem).wait()

    def zfill(i, _):
        valw[0, pl.ds(i * 16, 16)] = z16
        return 0
    lax.fori_loop(0, WSZ // 16, zfill, 0)
    for t in range(SLICE // WSZ):
        pltpu.sync_copy(valw.at[0],
                        spmem.at[pl.ds(s * SLICE + t * WSZ, WSZ)])
    plsc.subcore_barrier()

    prefetch(0)

    def window(w, _):
        s2 = w % 2
        s3 = w % 3

        @pl.when(w >= 2)
        def _():
            drain_scatters((w + 1) % 3)

        @pl.when(w + 1 < NWIN)
        def _():
            prefetch(w + 1)

        start = tile_base + w * WSZ
        pltpu.make_async_copy(flat_ref.at[pl.ds(start, WSZ)],
                              flatw.at[s2], in_sem).wait()
        pltpu.make_async_copy(val_ref.at[pl.ds(start, WSZ)],
                              valw.at[s3], in_sem).wait()

        def group(j, _):
            for k in range(8):
                e = j * 128 + k * 16
                flat = flatw[s2, pl.ds(e, 16)]
                rl = (flat >> 11) - row_base
                cl = (flat & (OUT_SIZE - 1)) - col_base
                mine = (rl.astype(jnp.uint32) < HALF_COLS) & (
                    cl.astype(jnp.uint32) < HALF_COLS)
                loc = jnp.where(mine, rl * HALF_COLS + cl,
                                CHUNK + (flat & (DUMP - 1)))
                locb[s3, j, pl.ds(k * 16, 16)] = loc
            return 0
        lax.fori_loop(0, JROWS, group, 0)

        def fire(j, _):
            pltpu.async_copy(valw.at[s3, pl.ds(j * 128, 128)],
                             spmem.at[locb.at[s3, j]], sem, add=True)
            return 0
        lax.fori_loop(0, JROWS, fire, 0)
        return 0
    lax.fori_loop(0, NWIN, window, 0)

    for t in range(2):
        drain_scatters(t)
    plsc.subcore_barrier()

    pltpu.sync_copy(spmem.at[pl.ds(s * SLICE, SLICE)],
                    out_ref.at[pl.ds(c * CHUNK + s * SLICE, SLICE)])
    plsc.subcore_barrier()


@functools.partial(jax.jit, static_argnums=0)
def _sc_scatter(col_base, flat, val):
    mesh = plsc.VectorSubcoreMesh(core_axis_name="c", subcore_axis_name="s")
    return pl.kernel(
        functools.partial(_sc_scatter_body, col_base),
        out_type=jax.ShapeDtypeStruct((IN_SIZE * HALF_COLS,), jnp.float32),
        mesh=mesh,
        scratch_types=[
            pltpu.VMEM_SHARED((CHUNK + DUMP,), jnp.float32),
            pltpu.VMEM((2, WSZ), jnp.int32),
            pltpu.VMEM((3, WSZ), jnp.float32),
            pltpu.VMEM((3, JROWS, 128), jnp.int32),
            pltpu.SemaphoreType.DMA,
            pltpu.SemaphoreType.DMA,
        ],
        compiler_params=pltpu.CompilerParams(needs_layout_passes=False),
        cost_estimate=pl.CostEstimate(flops=0, transcendentals=0,
                                      bytes_accessed=400_000_000),
    )(flat, val)


BM = 512


def _mm_body_first(x_ref, w_ref, b_ref, o_ref):
    acc = jax.lax.dot_general(x_ref[...], w_ref[...],
                              (((1,), (0,)), ((), ())),
                              preferred_element_type=jnp.float32)
    o_ref[...] = jnp.maximum(acc + b_ref[...], 0.0)


def _mm_body_second(x_ref, w_ref, b_ref, z_ref, o_ref):
    del z_ref
    acc = jax.lax.dot_general(x_ref[...], w_ref[...],
                              (((1,), (0,)), ((), ())),
                              preferred_element_type=jnp.float32)
    o_ref[...] = jnp.maximum(acc + b_ref[...], 0.0)


@jax.jit
def _matmul_half_first(x_bf, w_bf, bias_half):
    return pl.pallas_call(
        _mm_body_first,
        grid=(BATCH // BM,),
        in_specs=[
            pl.BlockSpec((BM, IN_SIZE), lambda i: (i, 0)),
            pl.BlockSpec((IN_SIZE, HALF_COLS), lambda i: (0, 0)),
            pl.BlockSpec((1, HALF_COLS), lambda i: (0, 0)),
        ],
        out_specs=pl.BlockSpec((BM, HALF_COLS), lambda i: (i, 0)),
        out_shape=jax.ShapeDtypeStruct((BATCH, OUT_SIZE), jnp.float32),
    )(x_bf, w_bf, bias_half)


@jax.jit
def _matmul_half_second(x_bf, w_bf, bias_half, z_prev):
    return pl.pallas_call(
        _mm_body_second,
        grid=(BATCH // BM,),
        in_specs=[
            pl.BlockSpec((BM, IN_SIZE), lambda i: (i, 0)),
            pl.BlockSpec((IN_SIZE, HALF_COLS), lambda i: (0, 0)),
            pl.BlockSpec((1, HALF_COLS), lambda i: (0, 0)),
            pl.BlockSpec(memory_space=pl.ANY),
        ],
        out_specs=pl.BlockSpec((BM, HALF_COLS), lambda i: (i, 1)),
        out_shape=jax.ShapeDtypeStruct((BATCH, OUT_SIZE), jnp.float32),
        input_output_aliases={3: 0},
    )(x_bf, w_bf, bias_half, z_prev)


def kernel(x, idx, val, bias):
    idx32 = idx.astype(jnp.int32)
    flat = idx32[:, 0] * OUT_SIZE + idx32[:, 1]
    x_bf = x.astype(jnp.bfloat16)
    bias2 = bias.reshape(2, HALF_COLS)

    wa = _sc_scatter(0, flat, val)
    wb = _sc_scatter(HALF_COLS, flat, val)
    wa_bf = wa.reshape(IN_SIZE, HALF_COLS).astype(jnp.bfloat16)
    wb_bf = wb.reshape(IN_SIZE, HALF_COLS).astype(jnp.bfloat16)

    z = _matmul_half_first(x_bf, wa_bf, bias2[0].reshape(1, HALF_COLS))
    z = _matmul_half_second(x_bf, wb_bf, bias2[1].reshape(1, HALF_COLS), z)
    return z
